# baseline (speedup 1.0000x reference)
import jax
import jax.numpy as jnp
from jax.experimental import pallas as pl
from jax.experimental.pallas import tpu as pltpu

ALPHA = 0.2
BLOCK = 8192


def _fused_body(x_ref, cache_ref, a_ref, b_ref, o_ref):
    x = x_ref[...]
    x16 = x.astype(jnp.bfloat16)
    cache = cache_ref[...]
    raw = jax.lax.dot_general(
        x16, cache, (((1,), (1,)), ((), ())), preferred_element_type=jnp.float32
    )
    ssq = jnp.sum(x * x, axis=1, keepdims=True)
    s = raw * jax.lax.rsqrt(jnp.maximum(jnp.broadcast_to(ssq, raw.shape), 1e-24))
    e = jnp.exp(s)
    p = (e * (1.0 / jnp.sum(e, axis=1, keepdims=True))).astype(jnp.bfloat16)
    o_ref[...] = (
        jnp.dot(x16, a_ref[...], preferred_element_type=jnp.float32)
        + jnp.dot(p, b_ref[...], preferred_element_type=jnp.float32)
    )


def kernel(text_token, cache, W):
    n_rows, d = text_token.shape
    m = cache.shape[0]
    a = (ALPHA * W[:, :d].T + jnp.eye(d, dtype=W.dtype)).astype(jnp.bfloat16)
    b = (cache @ (ALPHA * W[:, d:].T)).astype(jnp.bfloat16)
    cache = cache.astype(jnp.bfloat16)
    out = pl.pallas_call(
        _fused_body,
        grid=(n_rows // BLOCK,),
        in_specs=[
            pl.BlockSpec((BLOCK, d), lambda i: (i, 0)),
            pl.BlockSpec((m, d), lambda i: (0, 0)),
            pl.BlockSpec((d, d), lambda i: (0, 0)),
            pl.BlockSpec((m, d), lambda i: (0, 0)),
        ],
        out_specs=pl.BlockSpec((BLOCK, d), lambda i: (i, 0)),
        out_shape=jax.ShapeDtypeStruct((n_rows, d), text_token.dtype),
        compiler_params=pltpu.CompilerParams(
            dimension_semantics=("parallel",),
        ),
    )(text_token, cache, a, b)
    return (out, 0.0)


# speedup vs baseline: 1.1083x; 1.1083x over previous
import jax
import jax.numpy as jnp
from jax.experimental import pallas as pl
from jax.experimental.pallas import tpu as pltpu

ALPHA = 0.2
BLOCK = 16384


def _fused_body(x_ref, cache_ref, a_ref, b_ref, o_ref):
    x = x_ref[...]
    x16 = x.astype(jnp.bfloat16)
    cache = cache_ref[...]
    raw = jax.lax.dot_general(
        x16, cache, (((1,), (1,)), ((), ())), preferred_element_type=jnp.float32
    )
    ssq = jnp.sum(x * x, axis=1, keepdims=True)
    s = raw * jax.lax.rsqrt(jnp.maximum(jnp.broadcast_to(ssq, raw.shape), 1e-24))
    e = jnp.exp(s)
    corr = jnp.dot(e.astype(jnp.bfloat16), b_ref[...], preferred_element_type=jnp.float32)
    denom = jnp.sum(e, axis=1, keepdims=True)
    o_ref[...] = (
        jnp.dot(x16, a_ref[...], preferred_element_type=jnp.float32)
        + corr * (1.0 / denom)
    )


def kernel(text_token, cache, W):
    n_rows, d = text_token.shape
    m = cache.shape[0]
    a = (ALPHA * W[:, :d].T + jnp.eye(d, dtype=W.dtype)).astype(jnp.bfloat16)
    b = (cache @ (ALPHA * W[:, d:].T)).astype(jnp.bfloat16)
    cache = cache.astype(jnp.bfloat16)
    out = pl.pallas_call(
        _fused_body,
        grid=(n_rows // BLOCK,),
        in_specs=[
            pl.BlockSpec((BLOCK, d), lambda i: (i, 0)),
            pl.BlockSpec((m, d), lambda i: (0, 0)),
            pl.BlockSpec((d, d), lambda i: (0, 0)),
            pl.BlockSpec((m, d), lambda i: (0, 0)),
        ],
        out_specs=pl.BlockSpec((BLOCK, d), lambda i: (i, 0)),
        out_shape=jax.ShapeDtypeStruct((n_rows, d), text_token.dtype),
        compiler_params=pltpu.CompilerParams(
            dimension_semantics=("parallel",),
        ),
    )(text_token, cache, a, b)
    return (out, 0.0)


# speedup vs baseline: 1.1910x; 1.0746x over previous
import jax
import jax.numpy as jnp
from jax.experimental import pallas as pl
from jax.experimental.pallas import tpu as pltpu

ALPHA = 0.2
BLOCK = 16384


def _fused_body(x_ref, cache_ref, w_ref, o_ref):
    d = x_ref.shape[1]
    x = x_ref[...]
    x16 = x.astype(jnp.bfloat16)
    cache = cache_ref[...]
    cache16 = cache.astype(jnp.bfloat16)
    w1 = (ALPHA * w_ref[:, :d]).astype(jnp.bfloat16)
    w2 = (ALPHA * w_ref[:, d:]).astype(jnp.bfloat16)
    cb = jax.lax.dot_general(
        cache16, w2, (((1,), (1,)), ((), ())), preferred_element_type=jnp.float32
    ).astype(jnp.bfloat16)
    raw = jax.lax.dot_general(
        x16, cache16, (((1,), (1,)), ((), ())), preferred_element_type=jnp.float32
    )
    ssq = jnp.sum(x * x, axis=1, keepdims=True)
    s = raw * jax.lax.rsqrt(jnp.maximum(jnp.broadcast_to(ssq, raw.shape), 1e-24))
    e = jnp.exp(s)
    corr = jnp.dot(e.astype(jnp.bfloat16), cb, preferred_element_type=jnp.float32)
    denom = jnp.sum(e, axis=1, keepdims=True)
    o_ref[...] = (
        x
        + jax.lax.dot_general(
            x16, w1, (((1,), (1,)), ((), ())), preferred_element_type=jnp.float32
        )
        + corr * (1.0 / denom)
    )


def kernel(text_token, cache, W):
    n_rows, d = text_token.shape
    m = cache.shape[0]
    out = pl.pallas_call(
        _fused_body,
        grid=(n_rows // BLOCK,),
        in_specs=[
            pl.BlockSpec((BLOCK, d), lambda i: (i, 0)),
            pl.BlockSpec((m, d), lambda i: (0, 0)),
            pl.BlockSpec((d, 2 * d), lambda i: (0, 0)),
        ],
        out_specs=pl.BlockSpec((BLOCK, d), lambda i: (i, 0)),
        out_shape=jax.ShapeDtypeStruct((n_rows, d), text_token.dtype),
        compiler_params=pltpu.CompilerParams(
            dimension_semantics=("parallel",),
        ),
    )(text_token, cache, W)
    return (out, 0.0)
